# baseline (speedup 1.0000x reference)
import jax
import jax.numpy as jnp
from jax import lax
from jax.experimental import pallas as pl
from jax.experimental.pallas import tpu as pltpu

INPUT_DIM = 8192
HIDDEN = 512
K = 256
BATCH = 128

BK = 1024
BN = 2048
NB1 = INPUT_DIM // BK
NB2 = INPUT_DIM // BN
P2 = NB1
P4 = NB1 + NB2
STEPS = NB1 + 2 * NB2

BISECT_ITERS = 11
BB = BATCH


def _body(clr_ref, w1_ref, b1_ref, w2_ref, b2_ref,
          imp_out_ref, gated_ref, h_ref, thr_ref, imp_s_ref, qpk_ref,
          clr_s_ref):
    j = pl.program_id(1)

    @pl.when(j < P2)
    def _():
        @pl.when(j == 0)
        def _():
            h_ref[...] = jnp.zeros_like(h_ref)
        blk = clr_ref[...]
        h_ref[...] += jnp.dot(blk, w1_ref[...],
                              preferred_element_type=jnp.float32)
        clr_s_ref[:, pl.ds(jnp.minimum(j, NB1 - 1) * BK, BK)] = blk

    @pl.when(j == P2)
    def _():
        x = h_ref[...] + b1_ref[...]
        h_ref[...] = 0.5 * x * (1.0 + lax.erf(x * 0.7071067811865476))

    @pl.when((j >= P2) & (j < P4))
    def _():
        nb = j - P2
        scores = jnp.dot(h_ref[...], w2_ref[...],
                         preferred_element_type=jnp.float32) + b2_ref[...]
        sig = jax.nn.sigmoid(scores)
        imp_out_ref[...] = sig
        imp_s_ref[:, pl.ds(nb * BN, BN)] = sig
        q = jnp.minimum(sig * 32768.0, 32767.0).astype(jnp.int32)
        col = (nb % 2) * BN
        @pl.when(nb < 2)
        def _():
            qpk_ref[:, pl.ds(col, BN)] = (q << 16) + jnp.int32(-2147450880)
        @pl.when(nb >= 2)
        def _():
            qpk_ref[:, pl.ds(col, BN)] += q

    @pl.when(j == P4)
    def _():
        lo0 = jnp.zeros((BB, 1), jnp.int32)
        hi0 = jnp.full((BB, 1), 32768, jnp.int32)

        def it(_, carry):
            lo, hi = carry
            mid = (lo + hi) >> 1
            u = qpk_ref[...] - (mid * 65537)
            g = (u >> 15) & jnp.int32(0x00010001)
            s = jnp.sum(g, axis=1, keepdims=True)
            cnt = (s >> 16) + (s & 0xFFFF)
            ge = cnt >= K
            return jnp.where(ge, mid, lo), jnp.where(ge, hi, mid)

        lo, _ = lax.fori_loop(0, BISECT_ITERS, it, (lo0, hi0))
        thr_ref[...] = jnp.broadcast_to(
            lo.astype(jnp.float32) * (1.0 / 32768.0), (BB, 128))

    @pl.when(j >= P4)
    def _():
        nb = j - P4
        imp = imp_s_ref[:, pl.ds(nb * BN, BN)]
        thr = thr_ref[:, 0:1]
        gated_ref[...] = (clr_s_ref[:, pl.ds(nb * BN, BN)]
                          * jax.nn.sigmoid(10.0 * (imp - thr)))


def kernel(clr, W1, b1, W2, b2):
    b1r = b1.reshape(1, HIDDEN)
    b2r = b2.reshape(1, INPUT_DIM)

    grid = (1, STEPS)
    in_specs = [
        pl.BlockSpec((BB, BK), lambda i, j: (0, jnp.minimum(j, NB1 - 1))),
        pl.BlockSpec((BK, HIDDEN), lambda i, j: (jnp.minimum(j, NB1 - 1), 0)),
        pl.BlockSpec((1, HIDDEN), lambda i, j: (0, 0)),
        pl.BlockSpec((HIDDEN, BN),
                     lambda i, j: (0, jnp.clip(j - P2, 0, NB2 - 1))),
        pl.BlockSpec((1, BN), lambda i, j: (0, jnp.clip(j - P2, 0, NB2 - 1))),
    ]
    out_specs = [
        pl.BlockSpec((BB, BN), lambda i, j: (i, jnp.clip(j - P2, 0, NB2 - 1))),
        pl.BlockSpec((BB, BN), lambda i, j: (i, jnp.clip(j - P4, 0, NB2 - 1))),
    ]

    imp, gated = pl.pallas_call(
        _body,
        grid=grid,
        in_specs=in_specs,
        out_specs=out_specs,
        out_shape=[
            jax.ShapeDtypeStruct((BATCH, INPUT_DIM), jnp.float32),
            jax.ShapeDtypeStruct((BATCH, INPUT_DIM), jnp.float32),
        ],
        scratch_shapes=[
            pltpu.VMEM((BB, HIDDEN), jnp.float32),
            pltpu.VMEM((BB, 128), jnp.float32),
            pltpu.VMEM((BB, INPUT_DIM), jnp.float32),
            pltpu.VMEM((BB, INPUT_DIM // 2), jnp.int32),
            pltpu.VMEM((BB, INPUT_DIM), jnp.float32),
        ],
        compiler_params=pltpu.CompilerParams(
            dimension_semantics=("parallel", "arbitrary"),
        ),
    )(clr, W1, b1r, W2, b2r)
    return (gated, imp)


# speedup vs baseline: 1.0850x; 1.0850x over previous
import jax
import jax.numpy as jnp
from jax import lax
from jax.experimental import pallas as pl
from jax.experimental.pallas import tpu as pltpu

INPUT_DIM = 8192
HIDDEN = 512
K = 256
BATCH = 128

BK = 4096
BN = 2048
NB1 = INPUT_DIM // BK
NB2 = INPUT_DIM // BN
P2 = NB1
P4 = NB1 + NB2
STEPS = NB1 + 2 * NB2

BISECT_ITERS = 11
BB = BATCH


def _body(clr_ref, w1_ref, b1_ref, w2_ref, b2_ref,
          imp_out_ref, gated_ref, h_ref, thr_ref, imp_s_ref, qpk_ref):
    j = pl.program_id(1)

    @pl.when(j < P2)
    def _():
        @pl.when(j == 0)
        def _():
            h_ref[...] = jnp.zeros_like(h_ref)
        kb = jnp.minimum(j, NB1 - 1)
        h_ref[...] += jnp.dot(clr_ref[:, pl.ds(kb * BK, BK)], w1_ref[...],
                              preferred_element_type=jnp.float32)

    @pl.when(j == P2)
    def _():
        x = h_ref[...] + b1_ref[...]
        h_ref[...] = 0.5 * x * (1.0 + lax.erf(x * 0.7071067811865476))

    @pl.when((j >= P2) & (j < P4))
    def _():
        nb = j - P2
        scores = jnp.dot(h_ref[...], w2_ref[...],
                         preferred_element_type=jnp.float32) + b2_ref[...]
        sig = jax.nn.sigmoid(scores)
        imp_out_ref[...] = sig
        imp_s_ref[:, pl.ds(nb * BN, BN)] = sig
        q = jnp.minimum(sig * 32768.0, 32767.0).astype(jnp.int32)
        col = (nb % 2) * BN
        @pl.when(nb < 2)
        def _():
            qpk_ref[:, pl.ds(col, BN)] = (q << 16) + jnp.int32(-2147450880)
        @pl.when(nb >= 2)
        def _():
            qpk_ref[:, pl.ds(col, BN)] += q

    @pl.when(j == P4)
    def _():
        lo0 = jnp.zeros((BB, 1), jnp.int32)
        hi0 = jnp.full((BB, 1), 32768, jnp.int32)

        def it(_, carry):
            lo, hi = carry
            mid = (lo + hi) >> 1
            u = qpk_ref[...] - (mid * 65537)
            g = (u >> 15) & jnp.int32(0x00010001)
            s = jnp.sum(g, axis=1, keepdims=True)
            cnt = (s >> 16) + (s & 0xFFFF)
            ge = cnt >= K
            return jnp.where(ge, mid, lo), jnp.where(ge, hi, mid)

        lo, _ = lax.fori_loop(0, BISECT_ITERS, it, (lo0, hi0))
        thr_ref[...] = jnp.broadcast_to(
            lo.astype(jnp.float32) * (1.0 / 32768.0), (BB, 128))

    @pl.when(j >= P4)
    def _():
        nb = j - P4
        imp = imp_s_ref[:, pl.ds(nb * BN, BN)]
        thr = thr_ref[:, 0:1]
        gated_ref[...] = (clr_ref[:, pl.ds(nb * BN, BN)]
                          * jax.nn.sigmoid(10.0 * (imp - thr)))


def kernel(clr, W1, b1, W2, b2):
    b1r = b1.reshape(1, HIDDEN)
    b2r = b2.reshape(1, INPUT_DIM)

    grid = (1, STEPS)
    in_specs = [
        pl.BlockSpec((BB, INPUT_DIM), lambda i, j: (0, 0)),
        pl.BlockSpec((BK, HIDDEN), lambda i, j: (jnp.minimum(j, NB1 - 1), 0)),
        pl.BlockSpec((1, HIDDEN), lambda i, j: (0, 0)),
        pl.BlockSpec((HIDDEN, BN),
                     lambda i, j: (0, jnp.clip(j - P2, 0, NB2 - 1))),
        pl.BlockSpec((1, BN), lambda i, j: (0, jnp.clip(j - P2, 0, NB2 - 1))),
    ]
    out_specs = [
        pl.BlockSpec((BB, BN), lambda i, j: (i, jnp.clip(j - P2, 0, NB2 - 1))),
        pl.BlockSpec((BB, BN), lambda i, j: (i, jnp.clip(j - P4, 0, NB2 - 1))),
    ]

    imp, gated = pl.pallas_call(
        _body,
        grid=grid,
        in_specs=in_specs,
        out_specs=out_specs,
        out_shape=[
            jax.ShapeDtypeStruct((BATCH, INPUT_DIM), jnp.float32),
            jax.ShapeDtypeStruct((BATCH, INPUT_DIM), jnp.float32),
        ],
        scratch_shapes=[
            pltpu.VMEM((BB, HIDDEN), jnp.float32),
            pltpu.VMEM((BB, 128), jnp.float32),
            pltpu.VMEM((BB, INPUT_DIM), jnp.float32),
            pltpu.VMEM((BB, INPUT_DIM // 2), jnp.int32),
        ],
        compiler_params=pltpu.CompilerParams(
            dimension_semantics=("parallel", "arbitrary"),
        ),
    )(clr, W1, b1r, W2, b2r)
    return (gated, imp)
